# baseline (speedup 1.0000x reference)
import functools

import jax
import jax.numpy as jnp
from jax import lax
from jax.experimental import pallas as pl
from jax.experimental.pallas import tpu as pltpu
from jax.experimental.pallas import tpu_sc as plsc

NC = 2
NS = 16
NW = NC * NS
L = 16
D = 512
B = 16
BLK = 64
ROWS = 32768
TBLK = 2048
TGRID = ROWS // TBLK
H = D // 2
CH = H // L


def _split_row(total):
    return (total * 2 // 3) // TBLK * TBLK


def _sc_partial_sums(stack_lengths, features):
    mesh = plsc.VectorSubcoreMesh(core_axis_name="c", subcore_axis_name="s")

    @functools.partial(
        pl.kernel,
        out_type=jax.ShapeDtypeStruct((NW, B, D), jnp.float32),
        mesh=mesh,
        scratch_types=[
            pltpu.VMEM((1, L), jnp.int32),
            pltpu.VMEM((2, BLK, D), jnp.float32),
            pltpu.VMEM((B + 1, D), jnp.float32),
            pltpu.SemaphoreType.DMA,
            pltpu.SemaphoreType.DMA,
        ],
        compiler_params=pltpu.CompilerParams(needs_layout_passes=False),
    )
    def k(lens_hbm, feat_hbm, out_hbm, lens_v, bufs, acc, sem0, sem1):
        cid = lax.axis_index("c")
        sid = lax.axis_index("s")
        wid = cid * NS + sid

        pltpu.sync_copy(lens_hbm, lens_v)
        lens = lens_v[0, :]
        ends = jnp.cumsum(lens)
        iota = lax.iota(jnp.int32, L)
        e = [jnp.max(jnp.where(iota == j, ends, 0)) for j in range(B)]
        total = e[B - 1]
        srow = _split_row(total)

        fb = srow // BLK
        nblk = (total + BLK - 1) // BLK
        nmine = (nblk - fb - wid + NW - 1) // NW
        sems = (sem0, sem1)

        def blk_base(i):
            return (fb + wid + i * NW) * BLK

        for b in range(2):
            @pl.when(nmine > b)
            def _(b=b):
                pltpu.async_copy(
                    feat_hbm.at[pl.ds(blk_base(b), BLK)], bufs.at[b], sems[b]
                )

        zero = jnp.zeros((L,), jnp.float32)

        def zero_row(r, carry):
            for c in range(D // L):
                acc[r, pl.ds(c * L, L)] = zero
            return carry

        lax.fori_loop(0, B + 1, zero_row, 0)

        def consume(i, buf):
            base = blk_base(i)

            def run_cond(st):
                return st[0] < BLK

            def run_body(st):
                r = st[0]
                row = base + r
                s = jnp.int32(0)
                re = jnp.int32(BLK)
                for j in range(B):
                    s = s + jnp.where(e[j] <= row, 1, 0)
                    ej_rel = e[j] - base
                    re = jnp.where(
                        jnp.logical_and(e[j] > row, ej_rel < re), ej_rel, re
                    )
                for h in range(2):
                    col0 = h * H

                    def inner(rr, vs):
                        return tuple(
                            vs[c] + buf[rr, pl.ds(col0 + c * L, L)]
                            for c in range(CH)
                        )

                    init = tuple(
                        jnp.zeros((L,), jnp.float32) for _ in range(CH)
                    )
                    vs = lax.fori_loop(r, re, inner, init)
                    for c in range(CH):
                        plsc.addupdate(
                            acc.at[s, pl.ds(col0 + c * L, L)], vs[c]
                        )
                return (re,)

            lax.while_loop(run_cond, run_body, (jnp.int32(0),))

        def pair_body(p, carry):
            for b in range(2):
                i = 2 * p + b

                @pl.when(i < nmine)
                def _(i=i, b=b):
                    pltpu.make_async_copy(
                        feat_hbm.at[pl.ds(0, BLK)], bufs.at[b], sems[b]
                    ).wait()
                    consume(i, bufs.at[b])

                    @pl.when(i + 2 < nmine)
                    def _():
                        pltpu.async_copy(
                            feat_hbm.at[pl.ds(blk_base(i + 2), BLK)],
                            bufs.at[b],
                            sems[b],
                        )
            return carry

        lax.fori_loop(0, (nmine + 1) // 2, pair_body, 0)

        pltpu.sync_copy(acc.at[pl.ds(0, B)], out_hbm.at[wid])

    return k(stack_lengths, features)


def _tc_partial_sums(stack_lengths, features):
    def body(lens_ref, feat_hbm, o_ref, vbufs, sem):
        i = pl.program_id(0)
        e = []
        t = jnp.int32(0)
        for j in range(B):
            t = t + lens_ref[0, j]
            e.append(t)
        srow = _split_row(e[B - 1])
        nact = srow // TBLK

        def start(step):
            slot = lax.rem(step, 2)
            pltpu.make_async_copy(
                feat_hbm.at[pl.ds(step * TBLK, TBLK)],
                vbufs.at[slot],
                sem.at[slot],
            ).start()

        @pl.when(i == 0)
        def _():
            o_ref[...] = jnp.zeros_like(o_ref)

            @pl.when(nact > 0)
            def _():
                start(0)

        @pl.when(i < nact)
        def _():
            @pl.when(i + 1 < nact)
            def _():
                start(i + 1)

            slot = lax.rem(i, 2)
            pltpu.make_async_copy(
                feat_hbm.at[pl.ds(0, TBLK)], vbufs.at[slot], sem.at[slot]
            ).wait()

            base = i * TBLK
            rows = (
                jax.lax.broadcasted_iota(jnp.int32, (B, TBLK), 1) + base
            )
            col = jax.lax.broadcasted_iota(jnp.int32, (B, 1), 0)
            ecol = jnp.zeros((B, 1), jnp.int32)
            scol = jnp.zeros((B, 1), jnp.int32)
            for j in range(B):
                ecol = ecol + jnp.where(col == j, e[j], 0)
                if j > 0:
                    scol = scol + jnp.where(col == j, e[j - 1], 0)
            mask = jnp.logical_and(rows >= scol, rows < ecol).astype(
                jnp.bfloat16
            )
            x = vbufs[slot]
            hi = x.astype(jnp.bfloat16)
            lo = (x - hi.astype(jnp.float32)).astype(jnp.bfloat16)
            dn = (((1,), (0,)), ((), ()))
            o_ref[...] += jax.lax.dot_general(
                mask, hi, dn, preferred_element_type=jnp.float32
            ) + jax.lax.dot_general(
                mask, lo, dn, preferred_element_type=jnp.float32
            )

    return pl.pallas_call(
        body,
        grid=(TGRID,),
        out_shape=jax.ShapeDtypeStruct((B, D), jnp.float32),
        in_specs=[
            pl.BlockSpec(memory_space=pltpu.SMEM),
            pl.BlockSpec(memory_space=pl.ANY),
        ],
        out_specs=pl.BlockSpec((B, D), lambda i: (0, 0)),
        scratch_shapes=[
            pltpu.VMEM((2, TBLK, D), jnp.float32),
            pltpu.SemaphoreType.DMA((2,)),
        ],
    )(stack_lengths, features)


def _tc_combine(stack_lengths, sc_partials, tc_partial):
    def body(lens_ref, p_ref, t_ref, o_ref):
        s = t_ref[...]
        for w in range(NW):
            s = s + p_ref[w]
        for i in range(B):
            ln = lens_ref[0, i].astype(jnp.float32)
            o_ref[pl.ds(i, 1), :] = s[i : i + 1, :] / ln

    return pl.pallas_call(
        body,
        out_shape=jax.ShapeDtypeStruct((B, D), jnp.float32),
        in_specs=[
            pl.BlockSpec(memory_space=pltpu.SMEM),
            pl.BlockSpec(memory_space=pltpu.VMEM),
            pl.BlockSpec(memory_space=pltpu.VMEM),
        ],
        out_specs=pl.BlockSpec(memory_space=pltpu.VMEM),
    )(stack_lengths, sc_partials, tc_partial)


def kernel(stack_lengths, features):
    sc_partials = _sc_partial_sums(stack_lengths, features)
    tc_partial = _tc_partial_sums(stack_lengths, features)
    return _tc_combine(stack_lengths, sc_partials, tc_partial)


# speedup vs baseline: 1.0941x; 1.0941x over previous
import functools

import jax
import jax.numpy as jnp
from jax import lax
from jax.experimental import pallas as pl
from jax.experimental.pallas import tpu as pltpu
from jax.experimental.pallas import tpu_sc as plsc

NC = 2
NS = 16
NW = NC * NS
L = 16
D = 512
B = 16
BLK = 64
ROWS = 32768
TBLK = 2048
TGRID = ROWS // TBLK
H = D // 2
CH = H // L


def _split_row(total):
    return (total * 7 // 8) // TBLK * TBLK


def _sc_partial_sums(stack_lengths, features):
    mesh = plsc.VectorSubcoreMesh(core_axis_name="c", subcore_axis_name="s")

    @functools.partial(
        pl.kernel,
        out_type=jax.ShapeDtypeStruct((NW, B, D), jnp.float32),
        mesh=mesh,
        scratch_types=[
            pltpu.VMEM((1, L), jnp.int32),
            pltpu.VMEM((2, BLK, D), jnp.float32),
            pltpu.VMEM((B + 1, D), jnp.float32),
            pltpu.SemaphoreType.DMA,
            pltpu.SemaphoreType.DMA,
        ],
        compiler_params=pltpu.CompilerParams(needs_layout_passes=False),
    )
    def k(lens_hbm, feat_hbm, out_hbm, lens_v, bufs, acc, sem0, sem1):
        cid = lax.axis_index("c")
        sid = lax.axis_index("s")
        wid = cid * NS + sid

        pltpu.sync_copy(lens_hbm, lens_v)
        lens = lens_v[0, :]
        ends = jnp.cumsum(lens)
        iota = lax.iota(jnp.int32, L)
        e = [jnp.max(jnp.where(iota == j, ends, 0)) for j in range(B)]
        total = e[B - 1]
        srow = _split_row(total)

        fb = srow // BLK
        nblk = (total + BLK - 1) // BLK
        nmine = (nblk - fb - wid + NW - 1) // NW
        sems = (sem0, sem1)

        def blk_base(i):
            return (fb + wid + i * NW) * BLK

        for b in range(2):
            @pl.when(nmine > b)
            def _(b=b):
                pltpu.async_copy(
                    feat_hbm.at[pl.ds(blk_base(b), BLK)], bufs.at[b], sems[b]
                )

        zero = jnp.zeros((L,), jnp.float32)

        def zero_row(r, carry):
            for c in range(D // L):
                acc[r, pl.ds(c * L, L)] = zero
            return carry

        lax.fori_loop(0, B + 1, zero_row, 0)

        def consume(i, buf):
            base = blk_base(i)

            def run_cond(st):
                return st[0] < BLK

            def run_body(st):
                r = st[0]
                row = base + r
                s = jnp.int32(0)
                re = jnp.int32(BLK)
                for j in range(B):
                    s = s + jnp.where(e[j] <= row, 1, 0)
                    ej_rel = e[j] - base
                    re = jnp.where(
                        jnp.logical_and(e[j] > row, ej_rel < re), ej_rel, re
                    )
                for h in range(2):
                    col0 = h * H

                    def inner(rr, vs):
                        return tuple(
                            vs[c] + buf[rr, pl.ds(col0 + c * L, L)]
                            for c in range(CH)
                        )

                    init = tuple(
                        jnp.zeros((L,), jnp.float32) for _ in range(CH)
                    )
                    vs = lax.fori_loop(r, re, inner, init)
                    for c in range(CH):
                        plsc.addupdate(
                            acc.at[s, pl.ds(col0 + c * L, L)], vs[c]
                        )
                return (re,)

            lax.while_loop(run_cond, run_body, (jnp.int32(0),))

        def pair_body(p, carry):
            for b in range(2):
                i = 2 * p + b

                @pl.when(i < nmine)
                def _(i=i, b=b):
                    pltpu.make_async_copy(
                        feat_hbm.at[pl.ds(0, BLK)], bufs.at[b], sems[b]
                    ).wait()
                    consume(i, bufs.at[b])

                    @pl.when(i + 2 < nmine)
                    def _():
                        pltpu.async_copy(
                            feat_hbm.at[pl.ds(blk_base(i + 2), BLK)],
                            bufs.at[b],
                            sems[b],
                        )
            return carry

        lax.fori_loop(0, (nmine + 1) // 2, pair_body, 0)

        pltpu.sync_copy(acc.at[pl.ds(0, B)], out_hbm.at[wid])

    return k(stack_lengths, features)


def _tc_partial_sums(stack_lengths, features):
    def body(lens_ref, feat_hbm, o_ref, vbufs, sem):
        i = pl.program_id(0)
        e = []
        t = jnp.int32(0)
        for j in range(B):
            t = t + lens_ref[0, j]
            e.append(t)
        srow = _split_row(e[B - 1])
        nact = srow // TBLK

        def start(step):
            slot = lax.rem(step, 2)
            pltpu.make_async_copy(
                feat_hbm.at[pl.ds(step * TBLK, TBLK)],
                vbufs.at[slot],
                sem.at[slot],
            ).start()

        @pl.when(i == 0)
        def _():
            o_ref[...] = jnp.zeros_like(o_ref)

            @pl.when(nact > 0)
            def _():
                start(0)

        @pl.when(i < nact)
        def _():
            @pl.when(i + 1 < nact)
            def _():
                start(i + 1)

            slot = lax.rem(i, 2)
            pltpu.make_async_copy(
                feat_hbm.at[pl.ds(0, TBLK)], vbufs.at[slot], sem.at[slot]
            ).wait()

            base = i * TBLK
            rows = (
                jax.lax.broadcasted_iota(jnp.int32, (B, TBLK), 1) + base
            )
            col = jax.lax.broadcasted_iota(jnp.int32, (B, 1), 0)
            ecol = jnp.zeros((B, 1), jnp.int32)
            scol = jnp.zeros((B, 1), jnp.int32)
            for j in range(B):
                ecol = ecol + jnp.where(col == j, e[j], 0)
                if j > 0:
                    scol = scol + jnp.where(col == j, e[j - 1], 0)
            mask = jnp.logical_and(rows >= scol, rows < ecol).astype(
                jnp.bfloat16
            )
            x = vbufs[slot]
            hi = x.astype(jnp.bfloat16)
            lo = (x - hi.astype(jnp.float32)).astype(jnp.bfloat16)
            dn = (((1,), (0,)), ((), ()))
            o_ref[...] += jax.lax.dot_general(
                mask, hi, dn, preferred_element_type=jnp.float32
            ) + jax.lax.dot_general(
                mask, lo, dn, preferred_element_type=jnp.float32
            )

    return pl.pallas_call(
        body,
        grid=(TGRID,),
        out_shape=jax.ShapeDtypeStruct((B, D), jnp.float32),
        in_specs=[
            pl.BlockSpec(memory_space=pltpu.SMEM),
            pl.BlockSpec(memory_space=pl.ANY),
        ],
        out_specs=pl.BlockSpec((B, D), lambda i: (0, 0)),
        scratch_shapes=[
            pltpu.VMEM((2, TBLK, D), jnp.float32),
            pltpu.SemaphoreType.DMA((2,)),
        ],
    )(stack_lengths, features)


def _tc_combine(stack_lengths, sc_partials, tc_partial):
    def body(lens_ref, p_ref, t_ref, o_ref):
        s = t_ref[...]
        for w in range(NW):
            s = s + p_ref[w]
        for i in range(B):
            ln = lens_ref[0, i].astype(jnp.float32)
            o_ref[pl.ds(i, 1), :] = s[i : i + 1, :] / ln

    return pl.pallas_call(
        body,
        out_shape=jax.ShapeDtypeStruct((B, D), jnp.float32),
        in_specs=[
            pl.BlockSpec(memory_space=pltpu.SMEM),
            pl.BlockSpec(memory_space=pltpu.VMEM),
            pl.BlockSpec(memory_space=pltpu.VMEM),
        ],
        out_specs=pl.BlockSpec(memory_space=pltpu.VMEM),
    )(stack_lengths, sc_partials, tc_partial)


def kernel(stack_lengths, features):
    sc_partials = _sc_partial_sums(stack_lengths, features)
    tc_partial = _tc_partial_sums(stack_lengths, features)
    return _tc_combine(stack_lengths, sc_partials, tc_partial)
